# baseline (speedup 1.0000x reference)
import functools

import jax
import jax.numpy as jnp
from jax import lax
from jax.experimental import pallas as pl
from jax.experimental.pallas import tpu as pltpu
from jax.experimental.pallas import tpu_sc as plsc

N = 10000
H = 128
E = 320000
G = 64
C = 16

NW = 32
NCH = 80
EPT = NCH * 128
EP = EPT * NW
CW = 128
NCHT = EPT // CW
NP = 10112
RPT = NP // 16
DW = 128

_CHUNKS = []
_off = 0
while _off < RPT:
    _CHUNKS.append((_off, min(128, RPT - _off)))
    _off += min(128, RPT - _off)


def _fill_rows(ref, n_rows, n_cols, value):
    vec = jnp.full((16,), value, jnp.float32)

    def body(i, _):
        for k in range(n_cols // 16):
            ref[i, pl.ds(k * 16, 16)] = vec
        return 0

    lax.fori_loop(0, n_rows, body, 0)


def _sc_deg_body(d_hbm, out_hbm, idx_v, val_v, acc_sh):
    c = lax.axis_index("c")
    s = lax.axis_index("s")
    wid = s * 2 + c
    base = s * RPT

    _fill_rows(val_v, 128, DW, 0.0)
    for off, sz in _CHUNKS:
        pltpu.sync_copy(val_v.at[pl.ds(0, sz)], acc_sh.at[pl.ds(base + off, sz)])
    plsc.subcore_barrier()

    pltpu.sync_copy(d_hbm.at[pl.ds(wid * NCH, NCH)], idx_v)
    _fill_rows(val_v, 128, DW, 1.0)

    def body(j, _):
        pltpu.sync_copy(val_v, acc_sh.at[idx_v.at[j]], add=True)
        return 0

    lax.fori_loop(0, NCH, body, 0)
    plsc.subcore_barrier()

    pltpu.sync_copy(acc_sh.at[pl.ds(base, RPT)], out_hbm.at[c, pl.ds(base, RPT)])


_NBUF = 2
_SLAB = 16
_PAIR = 2 * NCHT
_K0 = 48

_WCHUNKS = []
_off = 0
while _off < RPT:
    _WCHUNKS.append((_off, min(CW, RPT - _off)))
    _off += min(CW, RPT - _off)


def _sc_agg_body(hs_hbm, s_hbm, d_hbm, out_hbm, src_v, dst_v,
                 r0, r1, acc_sh, g0, g1, t0, t1):
    bufs = (r0, r1)
    gsem = (g0, g1)
    ssem = (t0, t1)
    c = lax.axis_index("c")
    s = lax.axis_index("s")
    base = s * RPT

    _fill_rows(r0, CW, H, 0.0)
    for off, sz in _WCHUNKS:
        pltpu.sync_copy(r0.at[pl.ds(0, sz)], acc_sh.at[pl.ds(base + off, sz)])
    plsc.subcore_barrier()

    my_start = s * _PAIR + jnp.where(c == 0, 0, _K0)
    n_slab = jnp.where(c == 0, _K0 // _SLAB, (_PAIR - _K0) // _SLAB)

    def slab_body(h, _):
        off = my_start + h * _SLAB
        pltpu.sync_copy(s_hbm.at[pl.ds(off, _SLAB)], src_v)
        pltpu.sync_copy(d_hbm.at[pl.ds(off, _SLAB)], dst_v)

        for b in range(_NBUF):
            pltpu.async_copy(hs_hbm.at[src_v.at[b]], bufs[b], gsem[b])

        def group(g, _):
            for b in range(_NBUF):
                j = g * _NBUF + b
                pltpu.make_async_copy(hs_hbm.at[src_v.at[j]], bufs[b], gsem[b]).wait()
                pltpu.async_copy(bufs[b], acc_sh.at[dst_v.at[j]], ssem[b], add=True)
            for b in range(_NBUF):
                j = g * _NBUF + b
                pltpu.make_async_copy(bufs[b], acc_sh.at[dst_v.at[j]], ssem[b]).wait()
                pltpu.async_copy(hs_hbm.at[src_v.at[j + _NBUF]], bufs[b], gsem[b])
            return 0

        lax.fori_loop(0, _SLAB // _NBUF - 1, group, 0)

        gl = _SLAB - _NBUF
        for b in range(_NBUF):
            j = gl + b
            pltpu.make_async_copy(hs_hbm.at[src_v.at[j]], bufs[b], gsem[b]).wait()
            pltpu.async_copy(bufs[b], acc_sh.at[dst_v.at[j]], ssem[b], add=True)
        for b in range(_NBUF):
            j = gl + b
            pltpu.make_async_copy(bufs[b], acc_sh.at[dst_v.at[j]], ssem[b]).wait()
        return 0

    lax.fori_loop(0, n_slab, slab_body, 0)
    plsc.subcore_barrier()

    pltpu.sync_copy(acc_sh.at[pl.ds(base, RPT)], out_hbm.at[c, pl.ds(base, RPT)])


@functools.lru_cache(maxsize=None)
def _sc_kernels():
    mesh = plsc.VectorSubcoreMesh(core_axis_name="c", subcore_axis_name="s")
    deg_k = pl.kernel(
        _sc_deg_body,
        mesh=mesh,
        out_type=jax.ShapeDtypeStruct((2, NP, DW), jnp.float32),
        scratch_types=[
            pltpu.VMEM((NCH, 128), jnp.int32),
            pltpu.VMEM((128, DW), jnp.float32),
            pltpu.VMEM_SHARED((NP, DW), jnp.float32),
        ],
    )
    agg_k = pl.kernel(
        _sc_agg_body,
        mesh=mesh,
        out_type=jax.ShapeDtypeStruct((2, NP, H), jnp.float32),
        scratch_types=[
            pltpu.VMEM((_SLAB, CW), jnp.int32),
            pltpu.VMEM((_SLAB, CW), jnp.int32),
            pltpu.VMEM((CW, H), jnp.float32),
            pltpu.VMEM((CW, H), jnp.float32),
            pltpu.VMEM_SHARED((NP, H), jnp.float32),
            pltpu.SemaphoreType.DMA,
            pltpu.SemaphoreType.DMA,
            pltpu.SemaphoreType.DMA,
            pltpu.SemaphoreType.DMA,
        ],
    )
    return deg_k, agg_k


def _dinv_from(dpa_ref):
    deg = dpa_ref[0, :N, 0:1] + dpa_ref[1, :N, 0:1] + 1.0
    return lax.rsqrt(deg)


def _bn_relu(z, g, be):
    m = jnp.mean(z, axis=0, keepdims=True)
    v = jnp.mean(z * z, axis=0, keepdims=True) - m * m
    return jnp.maximum((z - m) * lax.rsqrt(v + 1e-5) * g + be, 0.0)


def _pre_body(x_ref, w_ref, dpa_ref, hs_ref):
    dinv = _dinv_from(dpa_ref)
    h = jnp.dot(x_ref[...], w_ref[...], preferred_element_type=jnp.float32)
    hs_ref[...] = h * dinv


def _mid_body(agg_ref, hs_ref, dpa_ref, w_ref, g_ref, be_ref, hout_ref, hsn_ref):
    dinv = _dinv_from(dpa_ref)
    conv = dinv * (agg_ref[0, :N, :] + agg_ref[1, :N, :] + hs_ref[...])
    hout = _bn_relu(conv, g_ref[...], be_ref[...])
    hout_ref[...] = hout
    hsn_ref[...] = jnp.dot(hout, w_ref[...], preferred_element_type=jnp.float32) * dinv


def _fin_body(agg_ref, hs_ref, dpa_ref, g_ref, be_ref, h1_ref, h2_ref, b_ref,
              wl1_ref, gl_ref, bel_ref, wl2_ref, bl2_ref, out_ref):
    dinv = _dinv_from(dpa_ref)
    conv = dinv * (agg_ref[0, :N, :] + agg_ref[1, :N, :] + hs_ref[...])
    h3 = _bn_relu(conv, g_ref[...], be_ref[...])

    onehot = (b_ref[...] == lax.broadcasted_iota(jnp.int32, (N, G), 1)).astype(jnp.float32)
    dn = (((0,), (0,)), ((), ()))
    s1 = lax.dot_general(onehot, h1_ref[...], dn, preferred_element_type=jnp.float32)
    s2 = lax.dot_general(onehot, h2_ref[...], dn, preferred_element_type=jnp.float32)
    s3 = lax.dot_general(onehot, h3, dn, preferred_element_type=jnp.float32)
    cnt = jnp.sum(onehot, axis=0)[:, None]
    scale = 1.0 / jnp.clip(cnt, 1.0, None)
    pooled = jnp.concatenate([s1, s2, s3], axis=1) * scale

    z = jnp.dot(pooled, wl1_ref[...], preferred_element_type=jnp.float32)
    z = _bn_relu(z, gl_ref[...], bel_ref[...])
    o = jnp.dot(z, wl2_ref[...], preferred_element_type=jnp.float32) + bl2_ref[...]
    mx = jnp.max(o, axis=1, keepdims=True)
    out_ref[...] = o - mx - jnp.log(jnp.sum(jnp.exp(o - mx), axis=1, keepdims=True))


def _tc_pre(x, w, dpa):
    return pl.pallas_call(
        _pre_body,
        out_shape=jax.ShapeDtypeStruct((N, H), jnp.float32),
    )(x, w, dpa)


def _tc_mid(agg, hs, dpa, w_next, g, be):
    return pl.pallas_call(
        _mid_body,
        out_shape=(jax.ShapeDtypeStruct((N, H), jnp.float32),
                   jax.ShapeDtypeStruct((N, H), jnp.float32)),
    )(agg, hs, dpa, w_next, g, be)


def _tc_fin(agg, hs, dpa, g, be, h1, h2, b2d, wl1, gl, bel, wl2, bl2):
    return pl.pallas_call(
        _fin_body,
        out_shape=jax.ShapeDtypeStruct((G, C), jnp.float32),
    )(agg, hs, dpa, g, be, h1, h2, b2d, wl1, gl, bel, wl2, bl2)


def kernel(x, edge_index, batch, W1, b1, W2, b2, W3, b3, g1, be1, g2, be2,
           g3, be3, Wl1, bl1, gl, bel, Wl2, bl2):
    src = edge_index[0]
    dst = edge_index[1]
    s_pad = jnp.concatenate([src, jnp.zeros((EP - E,), jnp.int32)])
    d_pad = jnp.concatenate([dst, jnp.full((EP - E,), N, jnp.int32)])
    s2 = s_pad.reshape(EP // CW, CW)
    d2 = d_pad.reshape(EP // CW, CW)
    d128 = d_pad.reshape(EP // 128, 128)

    _sc_deg, _sc_agg = _sc_kernels()
    dpa = _sc_deg(d128)

    g1_, be1_ = g1.reshape(1, H), be1.reshape(1, H)
    g2_, be2_ = g2.reshape(1, H), be2.reshape(1, H)
    g3_, be3_ = g3.reshape(1, H), be3.reshape(1, H)
    gl_, bel_ = gl.reshape(1, H), bel.reshape(1, H)

    hs1 = _tc_pre(x, W1, dpa)
    agg1 = _sc_agg(hs1, s2, d2)
    h1, hs2 = _tc_mid(agg1, hs1, dpa, W2, g1_, be1_)
    agg2 = _sc_agg(hs2, s2, d2)
    h2, hs3 = _tc_mid(agg2, hs2, dpa, W3, g2_, be2_)
    agg3 = _sc_agg(hs3, s2, d2)
    return _tc_fin(agg3, hs3, dpa, g3_, be3_, h1, h2, batch.reshape(N, 1),
                   Wl1, gl_, bel_, Wl2, bl2.reshape(1, C))


# speedup vs baseline: 1.1052x; 1.1052x over previous
import functools

import jax
import jax.numpy as jnp
from jax import lax
from jax.experimental import pallas as pl
from jax.experimental.pallas import tpu as pltpu
from jax.experimental.pallas import tpu_sc as plsc

N = 10000
H = 128
E = 320000
G = 64
C = 16

NW = 32
NCH = 80
EPT = NCH * 128
EP = EPT * NW
CW = 128
NCHT = EPT // CW
NP = 10112
RPT = NP // 16
DW = 128

_CHUNKS = []
_off = 0
while _off < RPT:
    _CHUNKS.append((_off, min(128, RPT - _off)))
    _off += min(128, RPT - _off)


def _fill_rows(ref, n_rows, n_cols, value):
    vec = jnp.full((16,), value, jnp.float32)

    def body(i, _):
        for k in range(n_cols // 16):
            ref[i, pl.ds(k * 16, 16)] = vec
        return 0

    lax.fori_loop(0, n_rows, body, 0)


def _sc_deg_body(d_hbm, out_hbm, idx_v, val_v, acc_sh):
    c = lax.axis_index("c")
    s = lax.axis_index("s")
    wid = s * 2 + c
    base = s * RPT

    _fill_rows(val_v, 128, DW, 0.0)
    for off, sz in _CHUNKS:
        pltpu.sync_copy(val_v.at[pl.ds(0, sz)], acc_sh.at[pl.ds(base + off, sz)])
    plsc.subcore_barrier()

    pltpu.sync_copy(d_hbm.at[pl.ds(wid * NCH, NCH)], idx_v)
    _fill_rows(val_v, 128, DW, 1.0)

    def body(j, _):
        pltpu.sync_copy(val_v, acc_sh.at[idx_v.at[j]], add=True)
        return 0

    lax.fori_loop(0, NCH, body, 0)
    plsc.subcore_barrier()

    pltpu.sync_copy(acc_sh.at[pl.ds(base, RPT)], out_hbm.at[c, pl.ds(base, RPT)])


_NBUF = 2
_SLAB = 16
_PAIR = 2 * NCHT
_K0 = 96

_WCHUNKS = []
_off = 0
while _off < RPT:
    _WCHUNKS.append((_off, min(CW, RPT - _off)))
    _off += min(CW, RPT - _off)


def _sc_agg_body(hs_hbm, s_hbm, d_hbm, out_hbm, src_v, dst_v,
                 r0, r1, acc_sh, g0, g1, t0, t1):
    bufs = (r0, r1)
    gsem = (g0, g1)
    ssem = (t0, t1)
    c = lax.axis_index("c")
    s = lax.axis_index("s")
    base = s * RPT

    _fill_rows(r0, CW, H, 0.0)
    for off, sz in _WCHUNKS:
        pltpu.sync_copy(r0.at[pl.ds(0, sz)], acc_sh.at[pl.ds(base + off, sz)])
    plsc.subcore_barrier()

    my_start = s * _PAIR + jnp.where(c == 0, 0, _K0)
    n_slab = jnp.where(c == 0, _K0 // _SLAB, (_PAIR - _K0) // _SLAB)

    def slab_body(h, _):
        off = my_start + h * _SLAB
        pltpu.sync_copy(s_hbm.at[pl.ds(off, _SLAB)], src_v)
        pltpu.sync_copy(d_hbm.at[pl.ds(off, _SLAB)], dst_v)

        for b in range(_NBUF):
            pltpu.async_copy(hs_hbm.at[src_v.at[b]], bufs[b], gsem[b])

        def group(g, _):
            for b in range(_NBUF):
                j = g * _NBUF + b
                pltpu.make_async_copy(hs_hbm.at[src_v.at[j]], bufs[b], gsem[b]).wait()
                pltpu.async_copy(bufs[b], acc_sh.at[dst_v.at[j]], ssem[b], add=True)
            for b in range(_NBUF):
                j = g * _NBUF + b
                pltpu.make_async_copy(bufs[b], acc_sh.at[dst_v.at[j]], ssem[b]).wait()
                pltpu.async_copy(hs_hbm.at[src_v.at[j + _NBUF]], bufs[b], gsem[b])
            return 0

        lax.fori_loop(0, _SLAB // _NBUF - 1, group, 0)

        gl = _SLAB - _NBUF
        for b in range(_NBUF):
            j = gl + b
            pltpu.make_async_copy(hs_hbm.at[src_v.at[j]], bufs[b], gsem[b]).wait()
            pltpu.async_copy(bufs[b], acc_sh.at[dst_v.at[j]], ssem[b], add=True)
        for b in range(_NBUF):
            j = gl + b
            pltpu.make_async_copy(bufs[b], acc_sh.at[dst_v.at[j]], ssem[b]).wait()
        return 0

    lax.fori_loop(0, n_slab, slab_body, 0)
    plsc.subcore_barrier()

    pltpu.sync_copy(acc_sh.at[pl.ds(base, RPT)], out_hbm.at[c, pl.ds(base, RPT)])


@functools.lru_cache(maxsize=None)
def _sc_kernels():
    mesh = plsc.VectorSubcoreMesh(core_axis_name="c", subcore_axis_name="s")
    deg_k = pl.kernel(
        _sc_deg_body,
        mesh=mesh,
        out_type=jax.ShapeDtypeStruct((2, NP, DW), jnp.float32),
        scratch_types=[
            pltpu.VMEM((NCH, 128), jnp.int32),
            pltpu.VMEM((128, DW), jnp.float32),
            pltpu.VMEM_SHARED((NP, DW), jnp.float32),
        ],
    )
    agg_k = pl.kernel(
        _sc_agg_body,
        mesh=mesh,
        out_type=jax.ShapeDtypeStruct((2, NP, H), jnp.float32),
        scratch_types=[
            pltpu.VMEM((_SLAB, CW), jnp.int32),
            pltpu.VMEM((_SLAB, CW), jnp.int32),
            pltpu.VMEM((CW, H), jnp.float32),
            pltpu.VMEM((CW, H), jnp.float32),
            pltpu.VMEM_SHARED((NP, H), jnp.float32),
            pltpu.SemaphoreType.DMA,
            pltpu.SemaphoreType.DMA,
            pltpu.SemaphoreType.DMA,
            pltpu.SemaphoreType.DMA,
        ],
    )
    return deg_k, agg_k


def _dinv_from(dpa_ref):
    deg = dpa_ref[0, :N, 0:1] + dpa_ref[1, :N, 0:1] + 1.0
    return lax.rsqrt(deg)


def _bn_relu(z, g, be):
    m = jnp.mean(z, axis=0, keepdims=True)
    v = jnp.mean(z * z, axis=0, keepdims=True) - m * m
    return jnp.maximum((z - m) * lax.rsqrt(v + 1e-5) * g + be, 0.0)


def _pre_body(x_ref, w_ref, dpa_ref, hs_ref):
    dinv = _dinv_from(dpa_ref)
    h = jnp.dot(x_ref[...], w_ref[...], preferred_element_type=jnp.float32)
    hs_ref[...] = h * dinv


def _mid_body(agg_ref, hs_ref, dpa_ref, w_ref, g_ref, be_ref, hout_ref, hsn_ref):
    dinv = _dinv_from(dpa_ref)
    conv = dinv * (agg_ref[0, :N, :] + agg_ref[1, :N, :] + hs_ref[...])
    hout = _bn_relu(conv, g_ref[...], be_ref[...])
    hout_ref[...] = hout
    hsn_ref[...] = jnp.dot(hout, w_ref[...], preferred_element_type=jnp.float32) * dinv


def _fin_body(agg_ref, hs_ref, dpa_ref, g_ref, be_ref, h1_ref, h2_ref, b_ref,
              wl1_ref, gl_ref, bel_ref, wl2_ref, bl2_ref, out_ref):
    dinv = _dinv_from(dpa_ref)
    conv = dinv * (agg_ref[0, :N, :] + agg_ref[1, :N, :] + hs_ref[...])
    h3 = _bn_relu(conv, g_ref[...], be_ref[...])

    onehot = (b_ref[...] == lax.broadcasted_iota(jnp.int32, (N, G), 1)).astype(jnp.float32)
    dn = (((0,), (0,)), ((), ()))
    s1 = lax.dot_general(onehot, h1_ref[...], dn, preferred_element_type=jnp.float32)
    s2 = lax.dot_general(onehot, h2_ref[...], dn, preferred_element_type=jnp.float32)
    s3 = lax.dot_general(onehot, h3, dn, preferred_element_type=jnp.float32)
    cnt = jnp.sum(onehot, axis=0)[:, None]
    scale = 1.0 / jnp.clip(cnt, 1.0, None)
    pooled = jnp.concatenate([s1, s2, s3], axis=1) * scale

    z = jnp.dot(pooled, wl1_ref[...], preferred_element_type=jnp.float32)
    z = _bn_relu(z, gl_ref[...], bel_ref[...])
    o = jnp.dot(z, wl2_ref[...], preferred_element_type=jnp.float32) + bl2_ref[...]
    mx = jnp.max(o, axis=1, keepdims=True)
    out_ref[...] = o - mx - jnp.log(jnp.sum(jnp.exp(o - mx), axis=1, keepdims=True))


def _tc_pre(x, w, dpa):
    return pl.pallas_call(
        _pre_body,
        out_shape=jax.ShapeDtypeStruct((N, H), jnp.float32),
    )(x, w, dpa)


def _tc_mid(agg, hs, dpa, w_next, g, be):
    return pl.pallas_call(
        _mid_body,
        out_shape=(jax.ShapeDtypeStruct((N, H), jnp.float32),
                   jax.ShapeDtypeStruct((N, H), jnp.float32)),
    )(agg, hs, dpa, w_next, g, be)


def _tc_fin(agg, hs, dpa, g, be, h1, h2, b2d, wl1, gl, bel, wl2, bl2):
    return pl.pallas_call(
        _fin_body,
        out_shape=jax.ShapeDtypeStruct((G, C), jnp.float32),
    )(agg, hs, dpa, g, be, h1, h2, b2d, wl1, gl, bel, wl2, bl2)


def kernel(x, edge_index, batch, W1, b1, W2, b2, W3, b3, g1, be1, g2, be2,
           g3, be3, Wl1, bl1, gl, bel, Wl2, bl2):
    src = edge_index[0]
    dst = edge_index[1]
    s_pad = jnp.concatenate([src, jnp.zeros((EP - E,), jnp.int32)])
    d_pad = jnp.concatenate([dst, jnp.full((EP - E,), N, jnp.int32)])
    s2 = s_pad.reshape(EP // CW, CW)
    d2 = d_pad.reshape(EP // CW, CW)
    d128 = d_pad.reshape(EP // 128, 128)

    _sc_deg, _sc_agg = _sc_kernels()
    dpa = _sc_deg(d128)

    g1_, be1_ = g1.reshape(1, H), be1.reshape(1, H)
    g2_, be2_ = g2.reshape(1, H), be2.reshape(1, H)
    g3_, be3_ = g3.reshape(1, H), be3.reshape(1, H)
    gl_, bel_ = gl.reshape(1, H), bel.reshape(1, H)

    hs1 = _tc_pre(x, W1, dpa)
    agg1 = _sc_agg(hs1, s2, d2)
    h1, hs2 = _tc_mid(agg1, hs1, dpa, W2, g1_, be1_)
    agg2 = _sc_agg(hs2, s2, d2)
    h2, hs3 = _tc_mid(agg2, hs2, dpa, W3, g2_, be2_)
    agg3 = _sc_agg(hs3, s2, d2)
    return _tc_fin(agg3, hs3, dpa, g3_, be3_, h1, h2, batch.reshape(N, 1),
                   Wl1, gl_, bel_, Wl2, bl2.reshape(1, C))


# speedup vs baseline: 1.1759x; 1.0640x over previous
import functools

import jax
import jax.numpy as jnp
from jax import lax
from jax.experimental import pallas as pl
from jax.experimental.pallas import tpu as pltpu
from jax.experimental.pallas import tpu_sc as plsc

N = 10000
H = 128
E = 320000
G = 64
C = 16

NW = 32
NCH = 80
EPT = NCH * 128
EP = EPT * NW
CW = 128
NCHT = EPT // CW
NP = 10112
RPT = NP // 16
DW = 128

_CHUNKS = []
_off = 0
while _off < RPT:
    _CHUNKS.append((_off, min(128, RPT - _off)))
    _off += min(128, RPT - _off)


def _fill_rows(ref, n_rows, n_cols, value):
    vec = jnp.full((16,), value, jnp.float32)

    def body(i, _):
        for k in range(n_cols // 16):
            ref[i, pl.ds(k * 16, 16)] = vec
        return 0

    lax.fori_loop(0, n_rows, body, 0)


def _sc_deg_body(d_hbm, out_hbm, idx_v, val_v, acc_sh):
    c = lax.axis_index("c")
    s = lax.axis_index("s")
    wid = s * 2 + c
    base = s * RPT

    _fill_rows(val_v, 128, DW, 0.0)
    for off, sz in _CHUNKS:
        pltpu.sync_copy(val_v.at[pl.ds(0, sz)], acc_sh.at[pl.ds(base + off, sz)])
    plsc.subcore_barrier()

    pltpu.sync_copy(d_hbm.at[pl.ds(wid * NCH, NCH)], idx_v)
    _fill_rows(val_v, 128, DW, 1.0)

    def body(j, _):
        pltpu.sync_copy(val_v, acc_sh.at[idx_v.at[j]], add=True)
        return 0

    lax.fori_loop(0, NCH, body, 0)
    plsc.subcore_barrier()

    pltpu.sync_copy(acc_sh.at[pl.ds(base, RPT)], out_hbm.at[c, pl.ds(base, RPT)])


_NBUF = 2
_SLAB = 16
_PAIR = 2 * NCHT
_K0 = 112

_WCHUNKS = []
_off = 0
while _off < RPT:
    _WCHUNKS.append((_off, min(CW, RPT - _off)))
    _off += min(CW, RPT - _off)


def _sc_agg_body(hs_hbm, s_hbm, d_hbm, out_hbm, src_v, dst_v,
                 r0, r1, acc_sh, g0, g1, t0, t1):
    bufs = (r0, r1)
    gsem = (g0, g1)
    ssem = (t0, t1)
    c = lax.axis_index("c")
    s = lax.axis_index("s")
    base = s * RPT

    _fill_rows(r0, CW, H, 0.0)
    for off, sz in _WCHUNKS:
        pltpu.sync_copy(r0.at[pl.ds(0, sz)], acc_sh.at[pl.ds(base + off, sz)])
    plsc.subcore_barrier()

    my_start = s * _PAIR + jnp.where(c == 0, 0, _K0)
    n_slab = jnp.where(c == 0, _K0 // _SLAB, (_PAIR - _K0) // _SLAB)

    def slab_body(h, _):
        off = my_start + h * _SLAB
        pltpu.sync_copy(s_hbm.at[pl.ds(off, _SLAB)], src_v)
        pltpu.sync_copy(d_hbm.at[pl.ds(off, _SLAB)], dst_v)

        for b in range(_NBUF):
            pltpu.async_copy(hs_hbm.at[src_v.at[b]], bufs[b], gsem[b])

        def group(g, _):
            for b in range(_NBUF):
                j = g * _NBUF + b
                pltpu.make_async_copy(hs_hbm.at[src_v.at[j]], bufs[b], gsem[b]).wait()
                pltpu.async_copy(bufs[b], acc_sh.at[dst_v.at[j]], ssem[b], add=True)
            for b in range(_NBUF):
                j = g * _NBUF + b
                pltpu.make_async_copy(bufs[b], acc_sh.at[dst_v.at[j]], ssem[b]).wait()
                pltpu.async_copy(hs_hbm.at[src_v.at[j + _NBUF]], bufs[b], gsem[b])
            return 0

        lax.fori_loop(0, _SLAB // _NBUF - 1, group, 0)

        gl = _SLAB - _NBUF
        for b in range(_NBUF):
            j = gl + b
            pltpu.make_async_copy(hs_hbm.at[src_v.at[j]], bufs[b], gsem[b]).wait()
            pltpu.async_copy(bufs[b], acc_sh.at[dst_v.at[j]], ssem[b], add=True)
        for b in range(_NBUF):
            j = gl + b
            pltpu.make_async_copy(bufs[b], acc_sh.at[dst_v.at[j]], ssem[b]).wait()
        return 0

    lax.fori_loop(0, n_slab, slab_body, 0)
    plsc.subcore_barrier()

    pltpu.sync_copy(acc_sh.at[pl.ds(base, RPT)], out_hbm.at[c, pl.ds(base, RPT)])


@functools.lru_cache(maxsize=None)
def _sc_kernels():
    mesh = plsc.VectorSubcoreMesh(core_axis_name="c", subcore_axis_name="s")
    deg_k = pl.kernel(
        _sc_deg_body,
        mesh=mesh,
        out_type=jax.ShapeDtypeStruct((2, NP, DW), jnp.float32),
        scratch_types=[
            pltpu.VMEM((NCH, 128), jnp.int32),
            pltpu.VMEM((128, DW), jnp.float32),
            pltpu.VMEM_SHARED((NP, DW), jnp.float32),
        ],
    )
    agg_k = pl.kernel(
        _sc_agg_body,
        mesh=mesh,
        out_type=jax.ShapeDtypeStruct((2, NP, H), jnp.float32),
        scratch_types=[
            pltpu.VMEM((_SLAB, CW), jnp.int32),
            pltpu.VMEM((_SLAB, CW), jnp.int32),
            pltpu.VMEM((CW, H), jnp.float32),
            pltpu.VMEM((CW, H), jnp.float32),
            pltpu.VMEM_SHARED((NP, H), jnp.float32),
            pltpu.SemaphoreType.DMA,
            pltpu.SemaphoreType.DMA,
            pltpu.SemaphoreType.DMA,
            pltpu.SemaphoreType.DMA,
        ],
    )
    return deg_k, agg_k


def _dinv_from(dpa_ref):
    deg = dpa_ref[0, :N, 0:1] + dpa_ref[1, :N, 0:1] + 1.0
    return lax.rsqrt(deg)


def _bn_relu(z, g, be):
    m = jnp.mean(z, axis=0, keepdims=True)
    v = jnp.mean(z * z, axis=0, keepdims=True) - m * m
    return jnp.maximum((z - m) * lax.rsqrt(v + 1e-5) * g + be, 0.0)


def _pre_body(x_ref, w_ref, dpa_ref, hs_ref):
    dinv = _dinv_from(dpa_ref)
    h = jnp.dot(x_ref[...], w_ref[...], preferred_element_type=jnp.float32)
    hs_ref[...] = h * dinv


def _mid_body(agg_ref, hs_ref, dpa_ref, w_ref, g_ref, be_ref, hout_ref, hsn_ref):
    dinv = _dinv_from(dpa_ref)
    conv = dinv * (agg_ref[0, :N, :] + agg_ref[1, :N, :] + hs_ref[...])
    hout = _bn_relu(conv, g_ref[...], be_ref[...])
    hout_ref[...] = hout
    hsn_ref[...] = jnp.dot(hout, w_ref[...], preferred_element_type=jnp.float32) * dinv


def _fin_body(agg_ref, hs_ref, dpa_ref, g_ref, be_ref, h1_ref, h2_ref, b_ref,
              wl1_ref, gl_ref, bel_ref, wl2_ref, bl2_ref, out_ref):
    dinv = _dinv_from(dpa_ref)
    conv = dinv * (agg_ref[0, :N, :] + agg_ref[1, :N, :] + hs_ref[...])
    h3 = _bn_relu(conv, g_ref[...], be_ref[...])

    onehot = (b_ref[...] == lax.broadcasted_iota(jnp.int32, (N, G), 1)).astype(jnp.float32)
    dn = (((0,), (0,)), ((), ()))
    s1 = lax.dot_general(onehot, h1_ref[...], dn, preferred_element_type=jnp.float32)
    s2 = lax.dot_general(onehot, h2_ref[...], dn, preferred_element_type=jnp.float32)
    s3 = lax.dot_general(onehot, h3, dn, preferred_element_type=jnp.float32)
    cnt = jnp.sum(onehot, axis=0)[:, None]
    scale = 1.0 / jnp.clip(cnt, 1.0, None)
    pooled = jnp.concatenate([s1, s2, s3], axis=1) * scale

    z = jnp.dot(pooled, wl1_ref[...], preferred_element_type=jnp.float32)
    z = _bn_relu(z, gl_ref[...], bel_ref[...])
    o = jnp.dot(z, wl2_ref[...], preferred_element_type=jnp.float32) + bl2_ref[...]
    mx = jnp.max(o, axis=1, keepdims=True)
    out_ref[...] = o - mx - jnp.log(jnp.sum(jnp.exp(o - mx), axis=1, keepdims=True))


def _tc_pre(x, w, dpa):
    return pl.pallas_call(
        _pre_body,
        out_shape=jax.ShapeDtypeStruct((N, H), jnp.float32),
    )(x, w, dpa)


def _tc_mid(agg, hs, dpa, w_next, g, be):
    return pl.pallas_call(
        _mid_body,
        out_shape=(jax.ShapeDtypeStruct((N, H), jnp.float32),
                   jax.ShapeDtypeStruct((N, H), jnp.float32)),
    )(agg, hs, dpa, w_next, g, be)


def _tc_fin(agg, hs, dpa, g, be, h1, h2, b2d, wl1, gl, bel, wl2, bl2):
    return pl.pallas_call(
        _fin_body,
        out_shape=jax.ShapeDtypeStruct((G, C), jnp.float32),
    )(agg, hs, dpa, g, be, h1, h2, b2d, wl1, gl, bel, wl2, bl2)


def kernel(x, edge_index, batch, W1, b1, W2, b2, W3, b3, g1, be1, g2, be2,
           g3, be3, Wl1, bl1, gl, bel, Wl2, bl2):
    src = edge_index[0]
    dst = edge_index[1]
    s_pad = jnp.concatenate([src, jnp.zeros((EP - E,), jnp.int32)])
    d_pad = jnp.concatenate([dst, jnp.full((EP - E,), N, jnp.int32)])
    s2 = s_pad.reshape(EP // CW, CW)
    d2 = d_pad.reshape(EP // CW, CW)
    d128 = d_pad.reshape(EP // 128, 128)

    _sc_deg, _sc_agg = _sc_kernels()
    dpa = _sc_deg(d128)

    g1_, be1_ = g1.reshape(1, H), be1.reshape(1, H)
    g2_, be2_ = g2.reshape(1, H), be2.reshape(1, H)
    g3_, be3_ = g3.reshape(1, H), be3.reshape(1, H)
    gl_, bel_ = gl.reshape(1, H), bel.reshape(1, H)

    hs1 = _tc_pre(x, W1, dpa)
    agg1 = _sc_agg(hs1, s2, d2)
    h1, hs2 = _tc_mid(agg1, hs1, dpa, W2, g1_, be1_)
    agg2 = _sc_agg(hs2, s2, d2)
    h2, hs3 = _tc_mid(agg2, hs2, dpa, W3, g2_, be2_)
    agg3 = _sc_agg(hs3, s2, d2)
    return _tc_fin(agg3, hs3, dpa, g3_, be3_, h1, h2, batch.reshape(N, 1),
                   Wl1, gl_, bel_, Wl2, bl2.reshape(1, C))
